# baseline (speedup 1.0000x reference)
import functools

import jax
import jax.numpy as jnp
import numpy as np
from jax import lax
from jax.experimental import pallas as pl
from jax.experimental.pallas import tpu as pltpu
from jax.experimental.pallas import tpu_sc as plsc

NW = 32
GATHER_N = 128
CHUNK_ROWS = 1664


def _sc_gather(tab3, idxT, bsz, f, d):
    n_per_w = bsz // NW
    n_g = n_per_w // GATHER_N
    mesh = plsc.VectorSubcoreMesh(core_axis_name="c", subcore_axis_name="s")

    @functools.partial(
        pl.kernel, mesh=mesh,
        out_type=jax.ShapeDtypeStruct((bsz, f * d), jnp.float32),
        compiler_params=pltpu.CompilerParams(use_tc_tiling_on_sc=False),
        scratch_types=[
            pltpu.VMEM((f, n_per_w), jnp.int32),
            pltpu.VMEM((n_per_w, d), jnp.float32),
            pltpu.SemaphoreType.DMA,
        ],
    )
    def k(tab_hbm, idx_hbm, out_hbm, idx_v, rows_v, sem):
        wid = lax.axis_index("s") * 2 + lax.axis_index("c")
        s0 = wid * n_per_w
        pltpu.sync_copy(idx_hbm.at[:, pl.ds(s0, n_per_w)], idx_v)

        def field_body(fi, carry):
            cps = [
                pltpu.async_copy(
                    tab_hbm.at[fi].at[
                        idx_v.at[fi, pl.ds(g * GATHER_N, GATHER_N)]],
                    rows_v.at[pl.ds(g * GATHER_N, GATHER_N)],
                    sem,
                )
                for g in range(n_g)
            ]
            for cp in cps:
                cp.wait()
            pltpu.sync_copy(
                rows_v, out_hbm.at[pl.ds(s0, n_per_w), pl.ds(fi * d, d)])
            return carry

        lax.fori_loop(0, f, field_body, 0)

    return k(tab3, idxT)


def _tc_mlp(flat2, w1at, w1bft, b1c, w2t, b2c, w3t, b3c, bb, f, d):
    b = flat2.shape[0]
    h1 = w1at.shape[0]
    h2 = w2t.shape[0]

    def body(flat_ref, w1a_ref, w1b_ref, b1_ref, w2_ref, b2_ref, w3_ref,
             b3_ref, out_ref):
        flat_t = jnp.transpose(flat_ref[...])
        ft = flat_t.reshape(f, d, bb)
        gs = []
        for i in range(f):
            prod = ft * ft[i][None]
            gs.append(jnp.sum(prod, axis=1))
        gt = jnp.concatenate(gs, axis=0)
        h = jnp.dot(w1a_ref[...], flat_t, preferred_element_type=jnp.float32)
        h = h + jnp.dot(w1b_ref[...], gt, preferred_element_type=jnp.float32)
        h = jnp.maximum(h + b1_ref[...], 0.0)
        h = jnp.dot(w2_ref[...], h, preferred_element_type=jnp.float32)
        h = jnp.maximum(h + b2_ref[...], 0.0)
        o = jnp.dot(w3_ref[...], h, preferred_element_type=jnp.float32)
        out_ref[...] = o + b3_ref[...]

    return pl.pallas_call(
        body,
        grid=(b // bb,),
        in_specs=[
            pl.BlockSpec((bb, f * d), lambda i: (i, 0)),
            pl.BlockSpec((h1, f * d), lambda i: (0, 0)),
            pl.BlockSpec((h1, f * f), lambda i: (0, 0)),
            pl.BlockSpec((h1, 1), lambda i: (0, 0)),
            pl.BlockSpec((h2, h1), lambda i: (0, 0)),
            pl.BlockSpec((h2, 1), lambda i: (0, 0)),
            pl.BlockSpec((1, h2), lambda i: (0, 0)),
            pl.BlockSpec((1, 1), lambda i: (0, 0)),
        ],
        out_specs=pl.BlockSpec((1, bb), lambda i: (0, i)),
        out_shape=jax.ShapeDtypeStruct((1, b), jnp.float32),
    )(flat2, w1at, w1bft, b1c, w2t, b2c, w3t, b3c)


def kernel(indices, tables, W1, b1, W2, b2, W3, b3):
    bsz, f = indices.shape
    _, v, d = tables.shape
    h1 = W1.shape[1]

    flat2 = _sc_gather(tables, indices.T, bsz, f, d)

    iu, ju = np.triu_indices(f, k=1)
    w1a = W1[: f * d]
    w1bf = jnp.zeros((f * f, h1), W1.dtype).at[iu * f + ju].set(W1[f * d:])

    out2 = _tc_mlp(flat2, w1a.T, w1bf.T, b1[:, None], W2.T, b2[:, None],
                   W3.T, b3[None, :], 512, f, d)
    return out2.reshape(bsz)


# speedup vs baseline: 1.2967x; 1.2967x over previous
import functools

import jax
import jax.numpy as jnp
import numpy as np
from jax import lax
from jax.experimental import pallas as pl
from jax.experimental.pallas import tpu as pltpu
from jax.experimental.pallas import tpu_sc as plsc

NW = 32
GATHER_N = 128
CHUNK_ROWS = 1664


def _sc_gather(tabT, idxT, bsz, f, d):
    n_planes = f * d
    per_w = n_planes // NW
    schunk = 4096
    n_sch = bsz // schunk
    mesh = plsc.VectorSubcoreMesh(core_axis_name="c", subcore_axis_name="s")

    @functools.partial(
        pl.kernel, mesh=mesh,
        out_type=jax.ShapeDtypeStruct((n_planes, bsz), jnp.float32),
        compiler_params=pltpu.CompilerParams(
            use_tc_tiling_on_sc=False, needs_layout_passes=False),
        scratch_types=[
            pltpu.VMEM((tabT.shape[2],), jnp.float32),
            pltpu.VMEM((schunk,), jnp.int32),
            pltpu.VMEM((schunk,), jnp.float32),
            pltpu.SemaphoreType.DMA,
        ],
    )
    def k(tab_hbm, idx_hbm, out_hbm, plane_v, idx_v, out_v, sem):
        wid = lax.axis_index("s") * 2 + lax.axis_index("c")

        def plane_body(j, carry):
            p = wid * per_w + j
            fi = p // d
            dd = p % d
            pltpu.sync_copy(tab_hbm.at[fi, dd, :], plane_v)

            def chunk_body(c, carry2):
                s0 = c * schunk
                pltpu.sync_copy(idx_hbm.at[fi, pl.ds(s0, schunk)], idx_v)

                def g_body(g, carry3):
                    sl = pl.ds(g * 16, 16)
                    out_v[sl] = plsc.load_gather(plane_v, [idx_v[sl]])
                    return carry3

                lax.fori_loop(0, schunk // 16, g_body, 0)
                pltpu.sync_copy(out_v, out_hbm.at[p, pl.ds(s0, schunk)])
                return carry2

            lax.fori_loop(0, n_sch, chunk_body, 0)
            return carry

        lax.fori_loop(0, per_w, plane_body, 0)

    return k(tabT, idxT)


def _tc_mlp(ft2, w1at, w1bft, b1c, w2t, b2c, w3t, b3c, bb, f, d):
    b = ft2.shape[1]
    h1 = w1at.shape[0]
    h2 = w2t.shape[0]

    def body(flat_ref, w1a_ref, w1b_ref, b1_ref, w2_ref, b2_ref, w3_ref,
             b3_ref, out_ref):
        flat_t = flat_ref[...]
        ft = flat_t.reshape(f, d, bb)
        gs = []
        for i in range(f):
            prod = ft * ft[i][None]
            gs.append(jnp.sum(prod, axis=1))
        gt = jnp.concatenate(gs, axis=0)
        h = jnp.dot(w1a_ref[...], flat_t, preferred_element_type=jnp.float32)
        h = h + jnp.dot(w1b_ref[...], gt, preferred_element_type=jnp.float32)
        h = jnp.maximum(h + b1_ref[...], 0.0)
        h = jnp.dot(w2_ref[...], h, preferred_element_type=jnp.float32)
        h = jnp.maximum(h + b2_ref[...], 0.0)
        o = jnp.dot(w3_ref[...], h, preferred_element_type=jnp.float32)
        out_ref[...] = o + b3_ref[...]

    return pl.pallas_call(
        body,
        grid=(b // bb,),
        in_specs=[
            pl.BlockSpec((f * d, bb), lambda i: (0, i)),
            pl.BlockSpec((h1, f * d), lambda i: (0, 0)),
            pl.BlockSpec((h1, f * f), lambda i: (0, 0)),
            pl.BlockSpec((h1, 1), lambda i: (0, 0)),
            pl.BlockSpec((h2, h1), lambda i: (0, 0)),
            pl.BlockSpec((h2, 1), lambda i: (0, 0)),
            pl.BlockSpec((1, h2), lambda i: (0, 0)),
            pl.BlockSpec((1, 1), lambda i: (0, 0)),
        ],
        out_specs=pl.BlockSpec((1, bb), lambda i: (0, i)),
        out_shape=jax.ShapeDtypeStruct((1, b), jnp.float32),
    )(ft2, w1at, w1bft, b1c, w2t, b2c, w3t, b3c)


def kernel(indices, tables, W1, b1, W2, b2, W3, b3):
    bsz, f = indices.shape
    _, v, d = tables.shape
    h1 = W1.shape[1]

    tabT = jnp.transpose(tables, (0, 2, 1))
    ft2 = _sc_gather(tabT, indices.T, bsz, f, d)

    iu, ju = np.triu_indices(f, k=1)
    w1a = W1[: f * d]
    w1bf = jnp.zeros((f * f, h1), W1.dtype).at[iu * f + ju].set(W1[f * d:])

    out2 = _tc_mlp(ft2, w1a.T, w1bf.T, b1[:, None], W2.T, b2[:, None],
                   W3.T, b3[None, :], 512, f, d)
    return out2.reshape(bsz)


# speedup vs baseline: 1.4750x; 1.1375x over previous
import functools

import jax
import jax.numpy as jnp
import numpy as np
from jax import lax
from jax.experimental import pallas as pl
from jax.experimental.pallas import tpu as pltpu
from jax.experimental.pallas import tpu_sc as plsc

NW = 32
GATHER_N = 128
CHUNK_ROWS = 1664


def _sc_gather(tabT, idxT, bsz, f, d):
    n_planes = f * d
    per_w = n_planes // NW
    schunk = 8192
    n_sch = bsz // schunk
    mesh = plsc.VectorSubcoreMesh(core_axis_name="c", subcore_axis_name="s")

    @functools.partial(
        pl.kernel, mesh=mesh,
        out_type=jax.ShapeDtypeStruct((n_planes, bsz), jnp.float32),
        compiler_params=pltpu.CompilerParams(
            use_tc_tiling_on_sc=False, needs_layout_passes=False),
        scratch_types=[
            pltpu.VMEM((tabT.shape[2],), jnp.float32),
            pltpu.VMEM((bsz,), jnp.int32),
            pltpu.VMEM((schunk,), jnp.float32),
            pltpu.SemaphoreType.DMA,
        ],
    )
    def k(tab_hbm, idx_hbm, out_hbm, plane_v, idx_v, out_v, sem):
        wid = lax.axis_index("s") * 2 + lax.axis_index("c")

        def plane_body(j, prev_fi):
            p = wid * per_w + j
            fi = p // d
            dd = p % d
            cp_plane = pltpu.async_copy(tab_hbm.at[fi, dd, :], plane_v, sem)

            @pl.when(fi != prev_fi)
            def _():
                pltpu.sync_copy(idx_hbm.at[fi, :], idx_v)

            cp_plane.wait()

            def chunk_body(c, carry2):
                s0 = c * schunk

                def g_body(g, carry3):
                    base = s0 + g * 64
                    for u in range(4):
                        sl = pl.ds(base + u * 16, 16)
                        osl = pl.ds(g * 64 + u * 16, 16)
                        out_v[osl] = plsc.load_gather(plane_v, [idx_v[sl]])
                    return carry3

                lax.fori_loop(0, schunk // 64, g_body, 0)
                pltpu.sync_copy(out_v, out_hbm.at[p, pl.ds(s0, schunk)])
                return carry2

            lax.fori_loop(0, n_sch, chunk_body, 0)
            return fi

        lax.fori_loop(0, per_w, plane_body, jnp.int32(-1))

    return k(tabT, idxT)


def _tc_mlp(ft2, w1at, w1bft, b1c, w2t, b2c, w3t, b3c, bb, f, d):
    b = ft2.shape[1]
    h1 = w1at.shape[0]
    h2 = w2t.shape[0]

    def body(flat_ref, w1a_ref, w1b_ref, b1_ref, w2_ref, b2_ref, w3_ref,
             b3_ref, out_ref):
        flat_t = flat_ref[...]
        ft = flat_t.reshape(f, d, bb)
        gs = []
        for i in range(f):
            prod = ft * ft[i][None]
            gs.append(jnp.sum(prod, axis=1))
        gt = jnp.concatenate(gs, axis=0)
        h = jnp.dot(w1a_ref[...], flat_t, preferred_element_type=jnp.float32)
        h = h + jnp.dot(w1b_ref[...], gt, preferred_element_type=jnp.float32)
        h = jnp.maximum(h + b1_ref[...], 0.0)
        h = jnp.dot(w2_ref[...], h, preferred_element_type=jnp.float32)
        h = jnp.maximum(h + b2_ref[...], 0.0)
        o = jnp.dot(w3_ref[...], h, preferred_element_type=jnp.float32)
        out_ref[...] = o + b3_ref[...]

    return pl.pallas_call(
        body,
        grid=(b // bb,),
        in_specs=[
            pl.BlockSpec((f * d, bb), lambda i: (0, i)),
            pl.BlockSpec((h1, f * d), lambda i: (0, 0)),
            pl.BlockSpec((h1, f * f), lambda i: (0, 0)),
            pl.BlockSpec((h1, 1), lambda i: (0, 0)),
            pl.BlockSpec((h2, h1), lambda i: (0, 0)),
            pl.BlockSpec((h2, 1), lambda i: (0, 0)),
            pl.BlockSpec((1, h2), lambda i: (0, 0)),
            pl.BlockSpec((1, 1), lambda i: (0, 0)),
        ],
        out_specs=pl.BlockSpec((1, bb), lambda i: (0, i)),
        out_shape=jax.ShapeDtypeStruct((1, b), jnp.float32),
    )(ft2, w1at, w1bft, b1c, w2t, b2c, w3t, b3c)


def kernel(indices, tables, W1, b1, W2, b2, W3, b3):
    bsz, f = indices.shape
    _, v, d = tables.shape
    h1 = W1.shape[1]

    tabT = jnp.transpose(tables, (0, 2, 1))
    ft2 = _sc_gather(tabT, indices.T, bsz, f, d)

    iu, ju = np.triu_indices(f, k=1)
    w1a = W1[: f * d]
    w1bf = jnp.zeros((f * f, h1), W1.dtype).at[iu * f + ju].set(W1[f * d:])

    out2 = _tc_mlp(ft2, w1a.T, w1bf.T, b1[:, None], W2.T, b2[:, None],
                   W3.T, b3[None, :], 512, f, d)
    return out2.reshape(bsz)
